# baseline (speedup 1.0000x reference)
import functools

import jax
import jax.numpy as jnp
from jax import lax
from jax.experimental import pallas as pl
from jax.experimental.pallas import tpu as pltpu
from jax.experimental.pallas import tpu_sc as plsc

B = 4096
ND = 13
NS = 26
VOCAB = 100000
ED = 32

NW = 32
CHUNK = 128
TBL = ED * VOCAB
NP = 2
FPP = NS // NP
CPW = B * FPP // (NW * CHUNK)
EPW = CPW * CHUNK


def _sc_gather_body(table_hbm, wb_hbm, out_hbm, wb_v, il_v, rows_v, sem):
    wid = lax.axis_index("s") * 2 + lax.axis_index("c")
    pltpu.sync_copy(wb_hbm.at[wid, 0], wb_v)

    for j in range(CPW):
        def build(s, _, base=j * CHUNK):
            wb = wb_v[pl.ds(base + s * 16, 16)]
            for k in range(ED):
                il_v[pl.ds(k * CHUNK + s * 16, 16)] = wb + k * VOCAB
            return 0

        lax.fori_loop(0, CHUNK // 16, build, 0, unroll=False)

        def fire(k, _, cbase=j * CHUNK * ED):
            pltpu.async_copy(
                table_hbm.at[il_v.at[pl.ds(k * CHUNK, CHUNK)]],
                rows_v.at[pl.ds(cbase + k * CHUNK, CHUNK)], sem)
            return 0

        def drain(k, _, cbase=j * CHUNK * ED):
            pltpu.make_async_copy(
                table_hbm.at[il_v.at[pl.ds(k * CHUNK, CHUNK)]],
                rows_v.at[pl.ds(cbase + k * CHUNK, CHUNK)], sem).wait()
            return 0

        lax.fori_loop(0, ED, fire, 0, unroll=False)
        lax.fori_loop(0, ED, drain, 0, unroll=False)

    pltpu.sync_copy(rows_v, out_hbm.at[wid, 0])


def _sc_gather(table_flat, wb3d):
    mesh = plsc.VectorSubcoreMesh(core_axis_name="c", subcore_axis_name="s")
    k = functools.partial(
        pl.kernel,
        mesh=mesh,
        out_type=jax.ShapeDtypeStruct((NW, 1, EPW * ED), jnp.float32),
        scratch_types=[
            pltpu.VMEM((EPW,), jnp.int32),
            pltpu.VMEM((CHUNK * ED,), jnp.int32),
            pltpu.VMEM((EPW * ED,), jnp.float32),
            pltpu.SemaphoreType.DMA,
        ],
        compiler_params=pltpu.CompilerParams(use_tc_tiling_on_sc=False),
    )(_sc_gather_body)
    return k(table_flat, wb3d)


BM = 512


def _mlp_body(dense_ref, embed_a, embed_b, w0a, w0ba, w0bb, b0, w1, b1, w2,
              b2, w3, b3, out_ref):
    f32 = jnp.float32
    x0 = jnp.dot(dense_ref[...], w0a[...], preferred_element_type=f32)
    x0 += jnp.dot(embed_a[...], w0ba[...], preferred_element_type=f32)
    x0 += jnp.dot(embed_b[...], w0bb[...], preferred_element_type=f32)
    h = jnp.maximum(x0 + b0[...], 0.0)
    h = jnp.maximum(
        jnp.dot(h, w1[...], preferred_element_type=f32) + b1[...], 0.0)
    h = jnp.maximum(
        jnp.dot(h, w2[...], preferred_element_type=f32) + b2[...], 0.0)
    out_ref[...] = jnp.dot(h, w3[...], preferred_element_type=f32) + b3[...]


def _mlp(dense, embed_a, embed_b, w0a, w0ba, w0bb, b0, w1, b1, w2, b2, w3,
         b3):
    nb = B // BM
    full = lambda shape: pl.BlockSpec(shape, lambda i: (0, 0))
    return pl.pallas_call(
        _mlp_body,
        grid=(nb,),
        in_specs=[
            pl.BlockSpec((BM, ND), lambda i: (i, 0)),
            pl.BlockSpec((BM, FPP * ED), lambda i: (i, 0)),
            pl.BlockSpec((BM, FPP * ED), lambda i: (i, 0)),
            full(w0a.shape),
            full(w0ba.shape),
            full(w0bb.shape),
            full(b0.shape),
            full(w1.shape),
            full(b1.shape),
            full(w2.shape),
            full(b2.shape),
            full(w3.shape),
            full(b3.shape),
        ],
        out_specs=pl.BlockSpec((BM, 256), lambda i: (i, 0)),
        out_shape=jax.ShapeDtypeStruct((B, 256), jnp.float32),
    )(dense, embed_a, embed_b, w0a, w0ba, w0bb, b0, w1, b1, w2, b2, w3, b3)


def kernel(inputs, embed_tables, W0, b0, W1, b1, W2, b2, W3, b3):
    dense = inputs[:, :ND]
    idx = inputs[:, ND:].astype(jnp.int32)
    foff = jnp.arange(FPP, dtype=jnp.int32) * TBL

    embeds = []
    for p in range(NP):
        wb = (idx[:, p * FPP:(p + 1) * FPP] + foff).reshape(-1)
        wb3d = wb.reshape(NW, 1, EPW)
        table_flat = (embed_tables[p * FPP:(p + 1) * FPP]
                      .transpose(0, 2, 1).reshape(-1))
        rows = _sc_gather(table_flat, wb3d)
        embeds.append(
            rows.reshape(NW, CPW, ED, CHUNK).transpose(0, 1, 3, 2)
            .reshape(B, FPP * ED))

    w0a = W0[:ND]
    w0ba = W0[ND:ND + FPP * ED]
    w0bb = W0[ND + FPP * ED:]
    out = _mlp(dense, embeds[0], embeds[1], w0a, w0ba, w0bb,
               b0.reshape(1, -1), W1, b1.reshape(1, -1), W2,
               b2.reshape(1, -1), W3, b3.reshape(1, -1))
    return out


# speedup vs baseline: 1.2286x; 1.2286x over previous
import functools

import jax
import jax.numpy as jnp
from jax import lax
from jax.experimental import pallas as pl
from jax.experimental.pallas import tpu as pltpu
from jax.experimental.pallas import tpu_sc as plsc

B = 4096
ND = 13
NS = 26
VOCAB = 100000
ED = 32

NW = 32
CHUNK = 128
TOT = B * NS
CPW = TOT // (NW * CHUNK)
EPW = CPW * CHUNK
TBL = ED * VOCAB


def _sc_gather_body(table_hbm, wb_hbm, out_hbm, wb_v, il_v, rows_v, sem):
    wid = lax.axis_index("s") * 2 + lax.axis_index("c")
    pltpu.sync_copy(wb_hbm.at[wid, 0], wb_v)

    def build(j, buf):
        def body(s, _):
            wb = wb_v[pl.ds(j * CHUNK + s * 16, 16)]
            for k in range(ED):
                il_v[buf, pl.ds(k * CHUNK + s * 16, 16)] = wb + k * VOCAB
            return 0
        lax.fori_loop(0, CHUNK // 16, body, 0, unroll=False)

    def fire(j, buf):
        def body(k, _):
            pltpu.async_copy(
                table_hbm.at[il_v.at[buf, pl.ds(k * CHUNK, CHUNK)]],
                rows_v.at[pl.ds(j * CHUNK * ED + k * CHUNK, CHUNK)], sem)
            return 0
        lax.fori_loop(0, ED, body, 0, unroll=False)

    def drain(j, buf):
        def body(k, _):
            pltpu.make_async_copy(
                table_hbm.at[il_v.at[buf, pl.ds(k * CHUNK, CHUNK)]],
                rows_v.at[pl.ds(j * CHUNK * ED + k * CHUNK, CHUNK)],
                sem).wait()
            return 0
        lax.fori_loop(0, ED, body, 0, unroll=False)

    build(0, 0)
    fire(0, 0)
    for j in range(CPW - 1):
        build(j + 1, (j + 1) & 1)
        fire(j + 1, (j + 1) & 1)
        drain(j, j & 1)
    drain(CPW - 1, (CPW - 1) & 1)

    pltpu.sync_copy(rows_v, out_hbm.at[wid, 0])


def _sc_gather(table_flat, wb3d):
    mesh = plsc.VectorSubcoreMesh(core_axis_name="c", subcore_axis_name="s")
    k = functools.partial(
        pl.kernel,
        mesh=mesh,
        out_type=jax.ShapeDtypeStruct((NW, 1, EPW * ED), jnp.float32),
        scratch_types=[
            pltpu.VMEM((EPW,), jnp.int32),
            pltpu.VMEM((2, CHUNK * ED), jnp.int32),
            pltpu.VMEM((EPW * ED,), jnp.float32),
            pltpu.SemaphoreType.DMA,
        ],
        compiler_params=pltpu.CompilerParams(use_tc_tiling_on_sc=False),
    )(_sc_gather_body)
    return k(table_flat, wb3d)


BM = 512


def _mlp_body(dense_ref, embed_ref, w0a, w0b, b0, w1, b1, w2, b2, w3, b3,
              out_ref):
    f32 = jnp.float32
    x0 = jnp.dot(dense_ref[...], w0a[...], preferred_element_type=f32)
    x0 += jnp.dot(embed_ref[...], w0b[...], preferred_element_type=f32)
    h = jnp.maximum(x0 + b0[...], 0.0)
    h = jnp.maximum(
        jnp.dot(h, w1[...], preferred_element_type=f32) + b1[...], 0.0)
    h = jnp.maximum(
        jnp.dot(h, w2[...], preferred_element_type=f32) + b2[...], 0.0)
    out_ref[...] = jnp.dot(h, w3[...], preferred_element_type=f32) + b3[...]


def _mlp(dense, embed, w0a, w0b, b0, w1, b1, w2, b2, w3, b3):
    nb = B // BM
    full = lambda shape: pl.BlockSpec(shape, lambda i: (0, 0))
    return pl.pallas_call(
        _mlp_body,
        grid=(nb,),
        in_specs=[
            pl.BlockSpec((BM, ND), lambda i: (i, 0)),
            pl.BlockSpec((BM, NS * ED), lambda i: (i, 0)),
            full(w0a.shape),
            full(w0b.shape),
            full(b0.shape),
            full(w1.shape),
            full(b1.shape),
            full(w2.shape),
            full(b2.shape),
            full(w3.shape),
            full(b3.shape),
        ],
        out_specs=pl.BlockSpec((BM, 256), lambda i: (i, 0)),
        out_shape=jax.ShapeDtypeStruct((B, 256), jnp.float32),
    )(dense, embed, w0a, w0b, b0, w1, b1, w2, b2, w3, b3)


def kernel(inputs, embed_tables, W0, b0, W1, b1, W2, b2, W3, b3):
    dense = inputs[:, :ND]
    idx = inputs[:, ND:].astype(jnp.int32)
    wb = (idx + jnp.arange(NS, dtype=jnp.int32) * TBL).reshape(-1)
    wb3d = wb.reshape(NW, 1, EPW)
    table_flat = embed_tables.transpose(0, 2, 1).reshape(-1)

    rows = _sc_gather(table_flat, wb3d)
    embed = (rows.reshape(NW, CPW, ED, CHUNK).transpose(0, 1, 3, 2)
             .reshape(B, NS * ED))

    w0a = W0[:ND]
    w0b = W0[ND:]
    out = _mlp(dense, embed, w0a, w0b, b0.reshape(1, -1), W1,
               b1.reshape(1, -1), W2, b2.reshape(1, -1), W3,
               b3.reshape(1, -1))
    return out
